# baseline (speedup 1.0000x reference)
import dataclasses
import functools

import jax
import jax.numpy as jnp
from jax import lax
from jax.experimental import pallas as pl
from jax.experimental.pallas import tpu as pltpu
from jax.experimental.pallas import tpu_sc as plsc

H = 256
W = 320
K = 15
RATIO_Y = 720.0 / 256.0
RATIO_X = 1280.0 / 320.0
BIG = 2 ** 30


def _pool(x):
    zrow = jnp.zeros((1, H), jnp.float32)
    v = jnp.maximum(
        x, jnp.maximum(jnp.concatenate([x[1:, :], zrow], axis=0),
                       jnp.concatenate([zrow, x[:-1, :]], axis=0)))
    zcol = jnp.zeros((W, 1), jnp.float32)
    hm = jnp.maximum(
        v, jnp.maximum(jnp.concatenate([v[:, 1:], zcol], axis=1),
                       jnp.concatenate([zcol, v[:, :-1]], axis=1)))
    return jnp.where(x == hm, x, 0.0)


def _stage1(h0_ref, h1_ref, sz0_ref, sz1_ref,
            box5_ref, gidx_ref, ny8_ref, lnf_ref, nsv_ref, nsc_ref,
            s0_ref, s1_ref, r0_ref, r1_ref, y0_ref, y1_ref, idx_s, val_s):
    rowio = jax.lax.broadcasted_iota(jnp.int32, (W, 1), 0)
    laneio = jax.lax.broadcasted_iota(jnp.int32, (1, H), 1)
    lanefull = jax.lax.broadcasted_iota(jnp.int32, (W, H), 1)

    for h_ref, s_ref, r_ref, y_ref in ((h0_ref, s0_ref, r0_ref, y0_ref),
                                       (h1_ref, s1_ref, r1_ref, y1_ref)):
        p = _pool(h_ref[...])
        s_ref[...] = p
        rm = jnp.max(p, axis=1, keepdims=True)
        r_ref[...] = rm
        y_ref[...] = jnp.min(jnp.where(p == rm, lanefull, BIG),
                             axis=1, keepdims=True)

    def pick(s_ref, r_ref, y_ref, c, k):
        rmax = r_ref[...]
        m = jnp.max(rmax)
        idx = jnp.min(jnp.where(rmax == m, y_ref[...] * W + rowio, BIG))
        x_ = idx % W
        row = s_ref[pl.ds(x_, 1), :]
        newrow = jnp.where(laneio == idx // W, -1.0, row)
        s_ref[pl.ds(x_, 1), :] = newrow
        rm = jnp.max(newrow, axis=1, keepdims=True)
        r_ref[pl.ds(x_, 1), :] = rm
        y_ref[pl.ds(x_, 1), :] = jnp.min(
            jnp.where(newrow == rm, laneio, BIG), axis=1, keepdims=True)
        val_s[c, k] = m
        idx_s[c, k] = idx

    for k in range(K):
        pick(s0_ref, r0_ref, y0_ref, 0, k)
        pick(s1_ref, r1_ref, y1_ref, 1, k)

    box5_ref[...] = jnp.full((K, 5), -1.0, jnp.float32)
    gidx_ref[...] = jnp.zeros((16, 8), jnp.int32)
    ny8_ref[...] = jnp.zeros((16, 8), jnp.int32)
    lnf_ref[...] = jnp.zeros((16, 8), jnp.float32)
    nsv_ref[...] = jnp.zeros((16, 1), jnp.float32)

    def f11(v):
        return jnp.full((1, 1), v, jnp.float32)

    jio8 = jax.lax.broadcasted_iota(jnp.int32, (1, 8), 1)
    par8 = jio8 % 2

    nb = jnp.int32(0)
    for k in range(K):
        score = val_s[0, k]
        bflat = idx_s[0, k]
        by = bflat // W
        bx = bflat % W
        sy = jnp.sum(jnp.where(laneio == by, sz0_ref[pl.ds(bx, 1), :], 0.0))
        sx = jnp.sum(jnp.where(laneio == by, sz1_ref[pl.ds(bx, 1), :], 0.0))
        byf = by.astype(jnp.float32)
        bxf = bx.astype(jnp.float32)
        tly = jnp.maximum(byf - sy * 0.5, 0.0) * RATIO_Y
        tlx = jnp.maximum(bxf - sx * 0.5, 0.0) * RATIO_X
        bry = jnp.minimum(byf + sy * 0.5, H - 1.0) * RATIO_Y
        brx = jnp.minimum(bxf + sx * 0.5, W - 1.0) * RATIO_X
        boxrow = jnp.concatenate(
            [f11(tly), f11(tlx), f11(bry), f11(brx), f11(score)], axis=1)
        bsel = score > 0.99

        @pl.when(bsel)
        def _():
            box5_ref[pl.ds(nb, 1), 0:5] = boxrow

        nb = nb + bsel.astype(jnp.int32)

        nflat = idx_s[1, k]
        ny = nflat // W
        nx = nflat % W
        gidx_ref[pl.ds(k, 1), :] = jnp.full((1, 8), nx * 8, jnp.int32) + jio8
        ny8_ref[pl.ds(k, 1), :] = jnp.full((1, 8), ny, jnp.int32)
        nyf = ny.astype(jnp.float32)
        nxf = nx.astype(jnp.float32)
        lnf_ref[pl.ds(k, 1), :] = jnp.where(
            par8 == 0, nyf * RATIO_Y, nxf * RATIO_X)
        nsv_ref[pl.ds(k, 1), :] = f11(val_s[1, k])
        nsc_ref[0, k] = val_s[1, k]


def _stage3(ov_ref, lnf_ref, nsv_ref, box5_ref, nsc_ref, out_ref, lrow_ref):
    out_ref[...] = jnp.full((K, 16), -1.0, jnp.float32)
    out_ref[:, 0:5] = box5_ref[...]
    par8 = jax.lax.broadcasted_iota(jnp.int32, (1, 8), 1) % 2
    ry8 = jnp.where(par8 == 0, RATIO_Y, RATIO_X)
    enm = lnf_ref[...] - ov_ref[...] * ry8
    lrow_ref[...] = jnp.concatenate(
        [enm[0:K, 0:4], lnf_ref[0:K, 0:2], enm[0:K, 4:8], nsv_ref[0:K, :]],
        axis=1)
    nn = jnp.int32(0)
    for k in range(K):
        nscore = nsc_ref[0, k]
        nsel = nscore > 0.5

        @pl.when(nsel)
        def _():
            out_ref[pl.ds(nn, 1), 5:16] = lrow_ref[pl.ds(k, 1), :]

        nn = nn + nsel.astype(jnp.int32)


_sc_params = pltpu.CompilerParams()
if "needs_layout_passes" in pltpu.CompilerParams.__dataclass_fields__:
    _sc_params = dataclasses.replace(_sc_params, needs_layout_passes=False)


@functools.partial(
    pl.kernel,
    out_type=jax.ShapeDtypeStruct((128,), jnp.float32),
    mesh=plsc.VectorSubcoreMesh(core_axis_name="c", subcore_axis_name="s"),
    compiler_params=_sc_params,
    scratch_types=[
        pltpu.VMEM((128,), jnp.int32),
        pltpu.VMEM((128,), jnp.int32),
        pltpu.VMEM((128, 256), jnp.float32),
        pltpu.VMEM((128,), jnp.float32),
        pltpu.SemaphoreType.DMA,
    ],
)
def _sc_gather(table_hbm, gidx_hbm, ny_hbm, ov_hbm,
               idx_v, ny_v, rows_v, ov_v, sem):
    @pl.when((lax.axis_index("c") == 0) & (lax.axis_index("s") == 0))
    def _():
        pltpu.sync_copy(gidx_hbm, idx_v)
        pltpu.sync_copy(ny_hbm, ny_v)
        pltpu.async_copy(table_hbm.at[idx_v], rows_v, sem).wait()
        io16 = lax.iota(jnp.int32, 16)
        for c in range(8):
            rowid = io16 + (16 * c)
            nyc = ny_v[pl.ds(16 * c, 16)]
            ov_v[pl.ds(16 * c, 16)] = plsc.load_gather(rows_v, [rowid, nyc])
        pltpu.sync_copy(ov_v, ov_hbm)


_stage1_call = pl.pallas_call(
    _stage1,
    out_shape=(
        jax.ShapeDtypeStruct((K, 5), jnp.float32),
        jax.ShapeDtypeStruct((16, 8), jnp.int32),
        jax.ShapeDtypeStruct((16, 8), jnp.int32),
        jax.ShapeDtypeStruct((16, 8), jnp.float32),
        jax.ShapeDtypeStruct((16, 1), jnp.float32),
        jax.ShapeDtypeStruct((1, K), jnp.float32),
    ),
    out_specs=(
        pl.BlockSpec(memory_space=pltpu.MemorySpace.VMEM),
        pl.BlockSpec(memory_space=pltpu.MemorySpace.VMEM),
        pl.BlockSpec(memory_space=pltpu.MemorySpace.VMEM),
        pl.BlockSpec(memory_space=pltpu.MemorySpace.VMEM),
        pl.BlockSpec(memory_space=pltpu.MemorySpace.VMEM),
        pl.BlockSpec(memory_space=pltpu.MemorySpace.SMEM),
    ),
    scratch_shapes=[
        pltpu.VMEM((W, H), jnp.float32),
        pltpu.VMEM((W, H), jnp.float32),
        pltpu.VMEM((W, 1), jnp.float32),
        pltpu.VMEM((W, 1), jnp.float32),
        pltpu.VMEM((W, 1), jnp.int32),
        pltpu.VMEM((W, 1), jnp.int32),
        pltpu.SMEM((2, K), jnp.int32),
        pltpu.SMEM((2, K), jnp.float32),
    ],
)

_stage3_call = pl.pallas_call(
    _stage3,
    out_shape=jax.ShapeDtypeStruct((K, 16), jnp.float32),
    in_specs=[
        pl.BlockSpec(memory_space=pltpu.MemorySpace.VMEM),
        pl.BlockSpec(memory_space=pltpu.MemorySpace.VMEM),
        pl.BlockSpec(memory_space=pltpu.MemorySpace.VMEM),
        pl.BlockSpec(memory_space=pltpu.MemorySpace.VMEM),
        pl.BlockSpec(memory_space=pltpu.MemorySpace.SMEM),
    ],
    scratch_shapes=[pltpu.VMEM((K, 11), jnp.float32)],
)


@jax.jit
def kernel(obj_heat_map, obj_offset_map, obj_size_maps):
    ht = jnp.transpose(obj_heat_map, (0, 2, 3, 1))
    st = jnp.transpose(obj_size_maps, (0, 2, 3, 1))
    table = jnp.transpose(obj_offset_map, (0, 2, 3, 1)).reshape(8 * W, H)
    h0 = ht[0, :, 0, :]
    h1 = ht[0, :, 1, :]
    s0 = st[0, :, 0, :]
    s1 = st[0, :, 1, :]
    box5, gidx, ny8, lnf, nsv, nsc = _stage1_call(h0, h1, s0, s1)
    ov = _sc_gather(table, gidx.reshape(128), ny8.reshape(128))
    return _stage3_call(ov.reshape(16, 8), lnf, nsv, box5, nsc)


# speedup vs baseline: 1.0294x; 1.0294x over previous
import dataclasses
import functools

import jax
import jax.numpy as jnp
from jax import lax
from jax.experimental import pallas as pl
from jax.experimental.pallas import tpu as pltpu
from jax.experimental.pallas import tpu_sc as plsc

H = 256
W = 320
K = 15
RATIO_Y = 720.0 / 256.0
RATIO_X = 1280.0 / 320.0
BIG = 2 ** 30


def _pool(x):
    zrow = jnp.zeros((1, H), jnp.float32)
    v = jnp.maximum(
        x, jnp.maximum(jnp.concatenate([x[1:, :], zrow], axis=0),
                       jnp.concatenate([zrow, x[:-1, :]], axis=0)))
    zcol = jnp.zeros((W, 1), jnp.float32)
    hm = jnp.maximum(
        v, jnp.maximum(jnp.concatenate([v[:, 1:], zcol], axis=1),
                       jnp.concatenate([zcol, v[:, :-1]], axis=1)))
    return jnp.where(x == hm, x, 0.0)


def _stage1(h0_ref, h1_ref, sz0_ref, sz1_ref,
            box5_ref, gidx_ref, ny8_ref, nidx_ref, nsc_ref,
            s0_ref, s1_ref, r0_ref, r1_ref, y0_ref, y1_ref, idx_s, val_s):
    rowio = jax.lax.broadcasted_iota(jnp.int32, (W, 1), 0)
    laneio = jax.lax.broadcasted_iota(jnp.int32, (1, H), 1)
    lanefull = jax.lax.broadcasted_iota(jnp.int32, (W, H), 1)

    for h_ref, s_ref, r_ref, y_ref in ((h0_ref, s0_ref, r0_ref, y0_ref),
                                       (h1_ref, s1_ref, r1_ref, y1_ref)):
        p = _pool(h_ref[...])
        s_ref[...] = p
        rm = jnp.max(p, axis=1, keepdims=True)
        r_ref[...] = rm
        y_ref[...] = jnp.min(jnp.where(p == rm, lanefull, BIG),
                             axis=1, keepdims=True)

    def pick(s_ref, r_ref, y_ref, c, k):
        rmax = r_ref[...]
        m = jnp.max(rmax)
        idx = jnp.min(jnp.where(rmax == m, y_ref[...] * W + rowio, BIG))
        x_ = idx % W
        row = s_ref[pl.ds(x_, 1), :]
        newrow = jnp.where(laneio == idx // W, -1.0, row)
        s_ref[pl.ds(x_, 1), :] = newrow
        rm = jnp.max(newrow, axis=1, keepdims=True)
        r_ref[pl.ds(x_, 1), :] = rm
        y_ref[pl.ds(x_, 1), :] = jnp.min(
            jnp.where(newrow == rm, laneio, BIG), axis=1, keepdims=True)
        val_s[c, k] = m
        idx_s[c, k] = idx

    for k in range(K):
        pick(s0_ref, r0_ref, y0_ref, 0, k)
        pick(s1_ref, r1_ref, y1_ref, 1, k)

    box5_ref[...] = jnp.full((K, 5), -1.0, jnp.float32)

    def f11(v):
        return jnp.full((1, 1), v, jnp.float32)

    jio8 = jax.lax.broadcasted_iota(jnp.int32, (1, 8), 1)

    gpieces = []
    nypieces = []
    nb = jnp.int32(0)
    for k in range(K):
        score = val_s[0, k]
        bflat = idx_s[0, k]
        by = bflat // W
        bx = bflat % W
        sy = jnp.sum(jnp.where(laneio == by, sz0_ref[pl.ds(bx, 1), :], 0.0))
        sx = jnp.sum(jnp.where(laneio == by, sz1_ref[pl.ds(bx, 1), :], 0.0))
        byf = by.astype(jnp.float32)
        bxf = bx.astype(jnp.float32)
        tly = jnp.maximum(byf - sy * 0.5, 0.0) * RATIO_Y
        tlx = jnp.maximum(bxf - sx * 0.5, 0.0) * RATIO_X
        bry = jnp.minimum(byf + sy * 0.5, H - 1.0) * RATIO_Y
        brx = jnp.minimum(bxf + sx * 0.5, W - 1.0) * RATIO_X
        boxrow = jnp.concatenate(
            [f11(tly), f11(tlx), f11(bry), f11(brx), f11(score)], axis=1)
        bsel = score > 0.99

        @pl.when(bsel)
        def _():
            box5_ref[pl.ds(nb, 1), 0:5] = boxrow

        nb = nb + bsel.astype(jnp.int32)

        nflat = idx_s[1, k]
        ny = nflat // W
        nx = nflat % W
        gpieces.append(jnp.full((1, 8), nx * 8, jnp.int32) + jio8)
        nypieces.append(jnp.full((1, 8), ny, jnp.int32))
        nsc_ref[0, k] = val_s[1, k]
        nidx_ref[0, k] = nflat

    gpieces.append(jnp.zeros((1, 8), jnp.int32))
    nypieces.append(jnp.zeros((1, 8), jnp.int32))
    gidx_ref[...] = jnp.concatenate(gpieces, axis=1)
    ny8_ref[...] = jnp.concatenate(nypieces, axis=1)


def _stage3(ov_ref, box5_ref, nidx_ref, nsc_ref, out_ref):
    out_ref[...] = jnp.full((K, 16), -1.0, jnp.float32)
    out_ref[:, 0:5] = box5_ref[...]
    par8 = jax.lax.broadcasted_iota(jnp.int32, (1, 8), 1) % 2
    ry8 = jnp.where(par8 == 0, RATIO_Y, RATIO_X)
    ov = ov_ref[...]

    def f11(v):
        return jnp.full((1, 1), v, jnp.float32)

    nn = jnp.int32(0)
    for k in range(K):
        nscore = nsc_ref[0, k]
        nflat = nidx_ref[0, k]
        lnfy = (nflat // W).astype(jnp.float32) * RATIO_Y
        lnfx = (nflat % W).astype(jnp.float32) * RATIO_X
        lnf8 = jnp.where(par8 == 0, lnfy, lnfx)
        enm = lnf8 - ov[:, 8 * k:8 * k + 8] * ry8
        lrow = jnp.concatenate(
            [enm[:, 0:4], f11(lnfy), f11(lnfx), enm[:, 4:8], f11(nscore)],
            axis=1)
        nsel = nscore > 0.5

        @pl.when(nsel)
        def _():
            out_ref[pl.ds(nn, 1), 5:16] = lrow

        nn = nn + nsel.astype(jnp.int32)


_sc_params = pltpu.CompilerParams()
if "needs_layout_passes" in pltpu.CompilerParams.__dataclass_fields__:
    _sc_params = dataclasses.replace(_sc_params, needs_layout_passes=False)


@functools.partial(
    pl.kernel,
    out_type=jax.ShapeDtypeStruct((128,), jnp.float32),
    mesh=plsc.VectorSubcoreMesh(core_axis_name="c", subcore_axis_name="s"),
    compiler_params=_sc_params,
    scratch_types=[
        pltpu.VMEM((128,), jnp.int32),
        pltpu.VMEM((128,), jnp.int32),
        pltpu.VMEM((128, 256), jnp.float32),
        pltpu.VMEM((128,), jnp.float32),
        pltpu.SemaphoreType.DMA,
    ],
)
def _sc_gather(table_hbm, gidx_hbm, ny_hbm, ov_hbm,
               idx_v, ny_v, rows_v, ov_v, sem):
    @pl.when((lax.axis_index("c") == 0) & (lax.axis_index("s") == 0))
    def _():
        pltpu.sync_copy(gidx_hbm, idx_v)
        pltpu.sync_copy(ny_hbm, ny_v)
        pltpu.async_copy(table_hbm.at[idx_v], rows_v, sem).wait()
        io16 = lax.iota(jnp.int32, 16)
        for c in range(8):
            rowid = io16 + (16 * c)
            nyc = ny_v[pl.ds(16 * c, 16)]
            ov_v[pl.ds(16 * c, 16)] = plsc.load_gather(rows_v, [rowid, nyc])
        pltpu.sync_copy(ov_v, ov_hbm)


_stage1_call = pl.pallas_call(
    _stage1,
    out_shape=(
        jax.ShapeDtypeStruct((K, 5), jnp.float32),
        jax.ShapeDtypeStruct((1, 128), jnp.int32),
        jax.ShapeDtypeStruct((1, 128), jnp.int32),
        jax.ShapeDtypeStruct((1, K), jnp.int32),
        jax.ShapeDtypeStruct((1, K), jnp.float32),
    ),
    out_specs=(
        pl.BlockSpec(memory_space=pltpu.MemorySpace.VMEM),
        pl.BlockSpec(memory_space=pltpu.MemorySpace.VMEM),
        pl.BlockSpec(memory_space=pltpu.MemorySpace.VMEM),
        pl.BlockSpec(memory_space=pltpu.MemorySpace.SMEM),
        pl.BlockSpec(memory_space=pltpu.MemorySpace.SMEM),
    ),
    scratch_shapes=[
        pltpu.VMEM((W, H), jnp.float32),
        pltpu.VMEM((W, H), jnp.float32),
        pltpu.VMEM((W, 1), jnp.float32),
        pltpu.VMEM((W, 1), jnp.float32),
        pltpu.VMEM((W, 1), jnp.int32),
        pltpu.VMEM((W, 1), jnp.int32),
        pltpu.SMEM((2, K), jnp.int32),
        pltpu.SMEM((2, K), jnp.float32),
    ],
)

_stage3_call = pl.pallas_call(
    _stage3,
    out_shape=jax.ShapeDtypeStruct((K, 16), jnp.float32),
    in_specs=[
        pl.BlockSpec(memory_space=pltpu.MemorySpace.VMEM),
        pl.BlockSpec(memory_space=pltpu.MemorySpace.VMEM),
        pl.BlockSpec(memory_space=pltpu.MemorySpace.SMEM),
        pl.BlockSpec(memory_space=pltpu.MemorySpace.SMEM),
    ],
)


@jax.jit
def kernel(obj_heat_map, obj_offset_map, obj_size_maps):
    ht = jnp.transpose(obj_heat_map, (0, 2, 3, 1))
    st = jnp.transpose(obj_size_maps, (0, 2, 3, 1))
    table = jnp.transpose(obj_offset_map, (0, 2, 3, 1)).reshape(8 * W, H)
    h0 = ht[0, :, 0, :]
    h1 = ht[0, :, 1, :]
    s0 = st[0, :, 0, :]
    s1 = st[0, :, 1, :]
    box5, gidx, ny8, nidx, nsc = _stage1_call(h0, h1, s0, s1)
    ov = _sc_gather(table, gidx.reshape(128), ny8.reshape(128))
    return _stage3_call(ov.reshape(1, 128), box5, nidx, nsc)
